# baseline (speedup 1.0000x reference)
import functools

import jax
import jax.numpy as jnp
from jax import lax
from jax.experimental import pallas as pl
from jax.experimental.pallas import tpu as pltpu
from jax.experimental.pallas import tpu_sc as plsc

_BN = 1024
_LANES = 256
_NC = 2
_NS = 16
_L = 16


def _tc_body(x1_ref, x2_ref, x3_ref, x4_ref, w_ref, eo_ref, ft_ref,
             accp_ref, acch_ref, *, ea, e):
    w = w_ref[...]
    dh = x1_ref.shape[1]
    acc = (jnp.dot(x1_ref[...], w[:dh], preferred_element_type=jnp.float32)
           + jnp.dot(x2_ref[...], w[dh:2 * dh], preferred_element_type=jnp.float32)
           + jnp.dot(x3_ref[...], w[2 * dh:3 * dh], preferred_element_type=jnp.float32)
           + jnp.dot(x4_ref[...], w[3 * dh:], preferred_element_type=jnp.float32))
    nc = 3 * e + 1
    accp_ref[...] = acc[:, :ea]
    acch_ref[0:nc] = acc[:, ea:ea + nc].T
    acch_ref[nc:nc + 1] = lax.bitcast_convert_type(eo_ref[0], jnp.float32)
    acch_ref[nc + 1:nc + 2] = lax.bitcast_convert_type(ft_ref[0], jnp.float32)


def _log_pos(x):
    bits = lax.bitcast_convert_type(x, jnp.int32)
    ex = (bits >> 23) - 127
    m = lax.bitcast_convert_type(
        (bits & 0x7FFFFF) | 0x3F800000, jnp.float32)
    z = (m - 1.0) / (m + 1.0)
    z2 = z * z
    ln_m = 2.0 * z * (1.0 + z2 * (1.0 / 3.0 + z2 * (0.2 + z2 * (1.0 / 7.0))))
    return ex.astype(jnp.float32) * 0.6931471805599453 + ln_m


def _sc_body(accp_hbm, acch_hbm, out_hbm,
             accp_v, acch_v, out_v, sem, *, e, a, nt):
    wid = lax.axis_index("s") * _NC + lax.axis_index("c")
    base = wid * nt
    cp1 = pltpu.make_async_copy(
        accp_hbm.at[pl.ds(base, nt), :], accp_v, sem)
    cp2 = pltpu.make_async_copy(
        acch_hbm.at[:, pl.ds(base, nt)], acch_v, sem)
    cp1.start()
    cp2.start()
    cp1.wait()
    cp2.wait()

    iota = lax.iota(jnp.int32, _L)
    for g in range(nt // _L):
        sl = pl.ds(g * _L, _L)
        cols = iota + (g * _L)

        m0 = acch_v[0, sl]
        mmax = m0
        marg = jnp.zeros((_L,), jnp.int32)
        ms = [m0]
        for f in range(1, e):
            mf = acch_v[f, sl]
            ms.append(mf)
            gt = mf > mmax
            marg = jnp.where(gt, f, marg)
            mmax = jnp.where(gt, mf, mmax)
        msum = jnp.zeros((_L,), jnp.float32)
        for mf in ms:
            msum = msum + jnp.exp(mf - mmax)
        mlp = -_log_pos(msum)
        mval = acch_v[e, sl]

        eo_g = plsc.bitcast(acch_v[3 * e + 1, sl], jnp.int32)
        ft_g = plsc.bitcast(acch_v[3 * e + 2, sl], jnp.int32)
        tlog = plsc.load_gather(acch_v, [eo_g + (e + 1), cols])
        tprob = 1.0 / (1.0 + jnp.exp(-tlog))
        req = (tprob > 0.5) | (ft_g > 0)
        newopt = jnp.where(req, marg, eo_g)
        tout = jnp.where(ft_g > 0, jnp.float32(0.0), tprob)
        val = plsc.load_gather(acch_v, [newopt + (2 * e + 1), cols])

        cbase = newopt * a
        s0 = plsc.load_gather(accp_v, [cols, cbase])
        smax = s0
        sarg = jnp.zeros((_L,), jnp.int32)
        ss = [s0]
        for j in range(1, a):
            sj = plsc.load_gather(accp_v, [cols, cbase + j])
            ss.append(sj)
            gt = sj > smax
            sarg = jnp.where(gt, j, sarg)
            smax = jnp.where(gt, sj, smax)
        ssum = jnp.zeros((_L,), jnp.float32)
        for sj in ss:
            ssum = ssum + jnp.exp(sj - smax)
        lp = -_log_pos(ssum)

        out_v[0, sl] = plsc.bitcast(sarg, jnp.float32)
        out_v[1, sl] = val
        out_v[2, sl] = lp
        out_v[3, sl] = plsc.bitcast(newopt, jnp.float32)
        out_v[4, sl] = mval
        out_v[5, sl] = mlp
        out_v[6, sl] = tout

    pltpu.sync_copy(out_v, out_hbm.at[:, pl.ds(base, nt)])


def kernel(observation, first_transition, executing_option, Wm, Wmv, Wt, Wp, Wv):
    n, d = observation.shape
    e = Wm.shape[1]
    a = Wp.shape[2]
    ea = e * a
    nh = 32
    ncols = ea + 2 * e + 1 + e
    nblk = n // _BN
    nt = n // (_NC * _NS)

    wp_flat = jnp.transpose(Wp, (1, 0, 2)).reshape(d, ea)
    w_all = jnp.concatenate(
        [wp_flat, Wm, Wmv, Wt, Wv[..., 0].T,
         jnp.zeros((d, _LANES - ncols), jnp.float32)], axis=1)
    eo3 = executing_option.astype(jnp.int32).reshape(nblk, 1, _BN)
    ft3 = first_transition.astype(jnp.int32).reshape(nblk, 1, _BN)

    row_spec = pl.BlockSpec((1, 1, _BN), lambda i: (i, 0, 0))
    accp, acch = pl.pallas_call(
        functools.partial(_tc_body, ea=ea, e=e),
        grid=(nblk,),
        in_specs=[
            pl.BlockSpec((_BN, d // 4), lambda i: (i, 0)),
            pl.BlockSpec((_BN, d // 4), lambda i: (i, 1)),
            pl.BlockSpec((_BN, d // 4), lambda i: (i, 2)),
            pl.BlockSpec((_BN, d // 4), lambda i: (i, 3)),
            pl.BlockSpec((d, _LANES), lambda i: (0, 0)),
            row_spec,
            row_spec,
        ],
        out_specs=[
            pl.BlockSpec((_BN, ea), lambda i: (i, 0)),
            pl.BlockSpec((nh, _BN), lambda i: (0, i)),
        ],
        out_shape=[
            jax.ShapeDtypeStruct((n, ea), jnp.float32),
            jax.ShapeDtypeStruct((nh, n), jnp.float32),
        ],
        compiler_params=pltpu.CompilerParams(
            dimension_semantics=("arbitrary",)),
    )(observation, observation, observation, observation, w_all, eo3, ft3)

    f32, i32 = jnp.float32, jnp.int32
    sc = pl.kernel(
        functools.partial(_sc_body, e=e, a=a, nt=nt),
        mesh=plsc.VectorSubcoreMesh(core_axis_name="c", subcore_axis_name="s"),
        compiler_params=pltpu.CompilerParams(needs_layout_passes=False),
        out_type=[jax.ShapeDtypeStruct((7, n), f32)],
        scratch_types=[
            pltpu.VMEM((nt, ea), f32),
            pltpu.VMEM((nh, nt), f32),
            pltpu.VMEM((7, nt), f32),
            pltpu.SemaphoreType.DMA,
        ],
    )
    (out,) = sc(accp, acch)
    return (
        lax.bitcast_convert_type(out[0], i32),
        out[1],
        out[2],
        lax.bitcast_convert_type(out[3], i32),
        out[4],
        out[5],
        out[6],
    )
